# Initial kernel scaffold; baseline (speedup 1.0000x reference)
import jax
import jax.numpy as jnp
from jax.experimental import pallas as pl


def kernel(x, edge_index, W1, b1, W2, b2):
    raise NotImplementedError("write your pallas kernel here")


# speedup vs baseline: 27.8942x; 27.8942x over previous
import functools

import jax
import jax.numpy as jnp
from jax import lax
from jax.experimental import pallas as pl
from jax.experimental.pallas import tpu as pltpu
from jax.experimental.pallas import tpu_sc as plsc

N_NODES = 10000
D = 128

NC = 2
NS = 16
NW = NC * NS
CHUNK = 128
CPW = 80
SEG = 40
EPW = CPW * CHUNK
E_PAD = NW * EPW
ACC_ROWS = 10240
ROWS_PER_TILE = ACC_ROWS // NS
N_DUMMY = ACC_ROWS - N_NODES

@functools.cache
def _mesh():
    return plsc.VectorSubcoreMesh(
        core_axis_name="c", subcore_axis_name="s", num_cores=NC, num_subcores=NS)


def _zero_vmem(ref, nrows, width):
    z = jnp.zeros((16,), jnp.float32)

    def body(i, _):
        r = i // (width // 16)
        k = i % (width // 16)
        ref[r, pl.ds(k * 16, 16)] = z
        return 0

    lax.fori_loop(0, nrows * (width // 16), body, 0)


def _deg_body(dst_hbm, out_hbm, didx, ones_v, acc):
    c = lax.axis_index("c")
    s = lax.axis_index("s")
    wid = c * NS + s
    pltpu.sync_copy(dst_hbm.at[wid], didx)
    _zero_vmem(ones_v, CHUNK, 16)
    for k in range(ROWS_PER_TILE // CHUNK):
        pltpu.sync_copy(ones_v, acc.at[pl.ds(s * ROWS_PER_TILE + k * CHUNK, CHUNK)])

    def refill(i, _):
        ones_v[i, pl.ds(0, 16)] = jnp.ones((16,), jnp.float32)
        return 0

    lax.fori_loop(0, CHUNK, refill, 0)
    plsc.subcore_barrier()

    def body(j, _):
        pltpu.sync_copy(ones_v, acc.at[didx.at[j]], add=True)
        return 0

    lax.fori_loop(0, CPW, body, 0)
    plsc.subcore_barrier()
    pltpu.sync_copy(
        acc.at[pl.ds(s * ROWS_PER_TILE, ROWS_PER_TILE)],
        out_hbm.at[c, pl.ds(s * ROWS_PER_TILE, ROWS_PER_TILE)],
    )


@functools.cache
def _deg_call():
    return pl.kernel(
        _deg_body,
        out_type=jax.ShapeDtypeStruct((NC, ACC_ROWS, 16), jnp.float32),
        mesh=_mesh(),
        scratch_types=[
            pltpu.VMEM((CPW, CHUNK), jnp.int32),
            pltpu.VMEM((CHUNK, 16), jnp.float32),
            pltpu.VMEM_SHARED((ACC_ROWS, 16), jnp.float32),
        ],
    )


def _agg_body(h_hbm, src_hbm, dst_hbm, out_hbm, sidx, didx, rows0, rows1,
              acc, sem0, sem1):
    c = lax.axis_index("c")
    s = lax.axis_index("s")
    wid = c * NS + s
    _zero_vmem(rows0, CHUNK, D)
    for k in range(ROWS_PER_TILE // CHUNK):
        pltpu.sync_copy(rows0, acc.at[pl.ds(s * ROWS_PER_TILE + k * CHUNK, CHUNK)])
    plsc.subcore_barrier()

    for seg in range(CPW // SEG):
        pltpu.sync_copy(src_hbm.at[wid, pl.ds(seg * SEG, SEG)], sidx)
        pltpu.sync_copy(dst_hbm.at[wid, pl.ds(seg * SEG, SEG)], didx)
        pltpu.async_copy(h_hbm.at[sidx.at[0]], rows0, sem0)

        def body(g, _):
            j0 = g * 2
            pltpu.make_async_copy(h_hbm.at[sidx.at[j0]], rows0, sem0).wait()
            pltpu.async_copy(h_hbm.at[sidx.at[j0 + 1]], rows1, sem1)
            pltpu.sync_copy(rows0, acc.at[didx.at[j0]], add=True)
            pltpu.make_async_copy(h_hbm.at[sidx.at[j0 + 1]], rows1, sem1).wait()

            @pl.when(j0 + 2 < SEG)
            def _():
                pltpu.async_copy(h_hbm.at[sidx.at[j0 + 2]], rows0, sem0)

            pltpu.sync_copy(rows1, acc.at[didx.at[j0 + 1]], add=True)
            return 0

        lax.fori_loop(0, SEG // 2, body, 0)
    plsc.subcore_barrier()
    for k in range(ROWS_PER_TILE // CHUNK):
        pltpu.sync_copy(
            acc.at[pl.ds(s * ROWS_PER_TILE + k * CHUNK, CHUNK)],
            out_hbm.at[c, pl.ds(s * ROWS_PER_TILE + k * CHUNK, CHUNK)],
        )


@functools.cache
def _agg_call():
    return pl.kernel(
        _agg_body,
        out_type=jax.ShapeDtypeStruct((NC, ACC_ROWS, D), jnp.float32),
        mesh=_mesh(),
        scratch_types=[
            pltpu.VMEM((SEG, CHUNK), jnp.int32),
            pltpu.VMEM((SEG, CHUNK), jnp.int32),
            pltpu.VMEM((CHUNK, D), jnp.float32),
            pltpu.VMEM((CHUNK, D), jnp.float32),
            pltpu.VMEM_SHARED((ACC_ROWS, D), jnp.float32),
            pltpu.SemaphoreType.DMA,
            pltpu.SemaphoreType.DMA,
        ],
    )


_BLK = 1000


def _dinv_from_deg(deg_ref):
    deg = deg_ref[0, :, 0] + deg_ref[1, :, 0] + 1.0
    return lax.rsqrt(deg)


def _tc_b_body(x_ref, w_ref, deg_ref, o_ref):
    dinv = _dinv_from_deg(deg_ref)
    h = jnp.dot(x_ref[...], w_ref[...], preferred_element_type=jnp.float32)
    o_ref[...] = h * dinv[:, None]


def _tc_d_body(s_ref, hp_ref, deg_ref, w_ref, b_ref, o_ref):
    dinv = _dinv_from_deg(deg_ref)
    t = dinv[:, None] * (s_ref[0] + s_ref[1] + hp_ref[...]) + b_ref[...][None, :]
    z = jnp.maximum(t, 0.0)
    h = jnp.dot(z, w_ref[...], preferred_element_type=jnp.float32)
    o_ref[...] = h * dinv[:, None]


def _tc_f_body(s_ref, hp_ref, deg_ref, b_ref, o_ref):
    dinv = _dinv_from_deg(deg_ref)
    t = dinv[:, None] * (s_ref[0] + s_ref[1] + hp_ref[...]) + b_ref[...][None, :]
    m = jnp.max(t, axis=1, keepdims=True)
    lse = jnp.log(jnp.sum(jnp.exp(t - m), axis=1, keepdims=True)) + m
    o_ref[...] = t - lse


_row_spec = pl.BlockSpec((_BLK, D), lambda i: (i, 0))
_deg_spec = pl.BlockSpec((NC, _BLK, 16), lambda i: (0, i, 0))
_s_spec = pl.BlockSpec((NC, _BLK, D), lambda i: (0, i, 0))
_w_spec = pl.BlockSpec((D, D), lambda i: (0, 0))
_b_spec = pl.BlockSpec((D,), lambda i: (0,))
_out_struct = jax.ShapeDtypeStruct((N_NODES, D), jnp.float32)

_tc_b = pl.pallas_call(
    _tc_b_body, grid=(N_NODES // _BLK,),
    in_specs=[_row_spec, _w_spec, _deg_spec],
    out_specs=_row_spec, out_shape=_out_struct)

_tc_d = pl.pallas_call(
    _tc_d_body, grid=(N_NODES // _BLK,),
    in_specs=[_s_spec, _row_spec, _deg_spec, _w_spec, _b_spec],
    out_specs=_row_spec, out_shape=_out_struct)

_tc_f = pl.pallas_call(
    _tc_f_body, grid=(N_NODES // _BLK,),
    in_specs=[_s_spec, _row_spec, _deg_spec, _b_spec],
    out_specs=_row_spec, out_shape=_out_struct)


def kernel(x, edge_index, W1, b1, W2, b2):
    n_edges = edge_index.shape[1]
    src = edge_index[0].astype(jnp.int32)
    dst = edge_index[1].astype(jnp.int32)
    pad = E_PAD - n_edges
    pad_i = jnp.arange(pad, dtype=jnp.int32)
    src_tab = jnp.concatenate([src, pad_i % N_NODES]).reshape(NW, CPW, CHUNK)
    dst_tab = jnp.concatenate([dst, N_NODES + pad_i % N_DUMMY]).reshape(NW, CPW, CHUNK)

    degp = _deg_call()(dst_tab)
    h1p = _tc_b(x, W1, degp)
    s1 = _agg_call()(h1p, src_tab, dst_tab)
    h2p = _tc_d(s1, h1p, degp, W2, b1)
    s2 = _agg_call()(h2p, src_tab, dst_tab)
    return _tc_f(s2, h2p, degp, b2)
